# baseline (speedup 1.0000x reference)
import jax
import jax.numpy as jnp
from jax.experimental import pallas as pl
from jax.experimental.pallas import tpu as pltpu

NE = 8
D_IN = 1024
D_HID = 512
CVLOSS_W = 0.01
SWITCHLOSS_W = 0.1
ZLOSS_W = 0.0001
N_TOK = 2048
N_ASSIGN = 2 * N_TOK
BG = 128
NB = 40
S_PAD = NB * BG


def _router_body(x_ref, wgt_ref, xbf_ref, ei_ref, gg_ref, loss_ref):
    xb = x_ref[...]
    xbf = xb.astype(jnp.bfloat16)
    xbf_ref[...] = xbf
    logits = jnp.dot(xbf, wgt_ref[...],
                     preferred_element_type=jnp.float32)
    lt = logits.T
    mx = jnp.max(lt, axis=0, keepdims=True)
    ex = jnp.exp(lt - mx)
    se = jnp.sum(ex, axis=0, keepdims=True)
    probs = ex / se
    m1 = jnp.max(probs, axis=0, keepdims=True)
    srow = jax.lax.broadcasted_iota(jnp.int32, probs.shape, 0)
    e1 = jnp.min(jnp.where(probs == m1, srow, NE), axis=0, keepdims=True)
    pwo = jnp.where(srow == e1, -jnp.inf, probs)
    m2 = jnp.max(pwo, axis=0, keepdims=True)
    e2 = jnp.min(jnp.where(pwo == m2, srow, NE), axis=0, keepdims=True)
    ei_ref[0:1, :] = e1
    ei_ref[1:2, :] = e2
    gg_ref[0:1, :] = m1
    gg_ref[1:2, :] = m2

    sel1 = srow == e1
    sel2 = srow == e2
    gsum = jnp.sum(jnp.where(sel1, m1, 0.0) + jnp.where(sel2, m2, 0.0),
                   axis=1, keepdims=True)
    cnt = jnp.sum(jnp.where(jnp.logical_and(sel1, m1 > 0), 1.0, 0.0)
                  + jnp.where(jnp.logical_and(sel2, m2 > 0), 1.0, 0.0),
                  axis=1, keepdims=True)
    psum = jnp.sum(probs, axis=1, keepdims=True)
    lse = mx + jnp.log(se)
    zsum = jnp.sum(lse * lse)

    w = gsum / jnp.maximum(jnp.sum(jnp.abs(gsum)), 1e-12)
    wm = jnp.mean(w)
    var = jnp.sum((w - wm) ** 2) / (NE - 1)
    cvloss = CVLOSS_W * var / (wm * wm + 1e-10)
    pn = psum / jnp.maximum(jnp.sum(jnp.abs(psum)), 1e-12)
    cn = cnt / jnp.maximum(jnp.sum(jnp.abs(cnt)), 1e-12)
    switchloss = SWITCHLOSS_W * (1.0 - jnp.sum(pn * cn)) * NE
    zloss = ZLOSS_W * zsum / N_TOK
    loss_ref[...] = (cvloss + switchloss + zloss).reshape(1, 1)


def _router(xf, wgt):
    return pl.pallas_call(
        _router_body,
        grid=(1,),
        in_specs=[
            pl.BlockSpec((N_TOK, D_IN), lambda i: (0, 0)),
            pl.BlockSpec((D_IN, NE), lambda i: (0, 0)),
        ],
        out_specs=[
            pl.BlockSpec((N_TOK, D_IN), lambda i: (0, 0)),
            pl.BlockSpec((2, N_TOK), lambda i: (0, 0)),
            pl.BlockSpec((2, N_TOK), lambda i: (0, 0)),
            pl.BlockSpec((1, 1), lambda i: (0, 0)),
        ],
        out_shape=[
            jax.ShapeDtypeStruct((N_TOK, D_IN), jnp.bfloat16),
            jax.ShapeDtypeStruct((2, N_TOK), jnp.int32),
            jax.ShapeDtypeStruct((2, N_TOK), jnp.float32),
            jax.ShapeDtypeStruct((1, 1), jnp.float32),
        ],
    )(xf, wgt.astype(jnp.bfloat16))


def _gmm_body(be_ref, xs_ref, w1_ref, w2_ref, out_ref):
    h = jnp.maximum(jnp.dot(xs_ref[...], w1_ref[0],
                            preferred_element_type=jnp.float32), 0.0)
    out_ref[...] = jnp.dot(h.astype(jnp.bfloat16), w2_ref[0],
                           preferred_element_type=jnp.float32)


def _gmm(be, xs, w1, w2):
    return pl.pallas_call(
        _gmm_body,
        grid_spec=pltpu.PrefetchScalarGridSpec(
            num_scalar_prefetch=1,
            grid=(NB,),
            in_specs=[
                pl.BlockSpec((BG, D_IN), lambda b, be: (b, 0)),
                pl.BlockSpec((1, D_IN, D_HID), lambda b, be: (be[b], 0, 0)),
                pl.BlockSpec((1, D_HID, D_IN), lambda b, be: (be[b], 0, 0)),
            ],
            out_specs=pl.BlockSpec((BG, D_IN), lambda b, be: (b, 0)),
        ),
        out_shape=jax.ShapeDtypeStruct((S_PAD, D_IN), jnp.float32),
        compiler_params=pltpu.CompilerParams(
            dimension_semantics=("arbitrary",),
        ),
    )(be, xs, w1, w2)


@jax.jit
def _moe_sparse(xf, wgt, w1, w2):
    xbf, ei, gg, loss = _router(xf, wgt)

    eflat = ei.reshape(-1)
    order = jnp.argsort(eflat, stable=True)
    es = eflat[order]
    counts = jnp.sum(eflat[None, :] == jnp.arange(NE)[:, None], axis=1)
    cum = jnp.cumsum(counts) - counts
    padc = ((counts + BG - 1) // BG) * BG
    base = jnp.cumsum(padc) - padc
    pend = base + padc
    i = jnp.arange(N_ASSIGN)
    slot_sorted = base[es] + i - cum[es]
    slot = jnp.zeros((N_ASSIGN,), jnp.int32).at[order].set(
        slot_sorted.astype(jnp.int32))
    row_of_slot = jnp.zeros((S_PAD,), jnp.int32).at[slot_sorted].set(
        (order % N_TOK).astype(jnp.int32))
    be = jnp.clip(jnp.sum(jnp.arange(NB)[:, None] * BG >= pend[None, :],
                          axis=1), 0, NE - 1).astype(jnp.int32)
    xs = xbf[row_of_slot]

    out = _gmm(be, xs, w1.astype(jnp.bfloat16), w2.astype(jnp.bfloat16))

    y = (out[slot[:N_TOK]] * gg[0][:, None]
         + out[slot[N_TOK:]] * gg[1][:, None])
    return y, loss


def kernel(x, Wg, W1, W2):
    bsz, length, emb = x.shape
    xf = x.reshape(-1, emb)
    y, loss = _moe_sparse(xf, Wg.T, W1, W2)
    return y.reshape(bsz, length, emb), loss[0, 0]
